# baseline (speedup 1.0000x reference)
import functools

import jax
import jax.numpy as jnp
from jax.experimental import pallas as pl
from jax.experimental.pallas import tpu as pltpu


def _mp_body(h_ref, a_ref, et_ref, ws_ref, wn_ref, b_ref, o_ref, *, d):
    h = h_ref[0]
    a = a_ref[0]
    et = et_ref[0]

    hs = jnp.dot(h, ws_ref[...], preferred_element_type=jnp.float32)
    hn = jnp.dot(h, wn_ref[:d], preferred_element_type=jnp.float32)
    deg = jnp.sum(a, axis=1, keepdims=True)
    m = jnp.sum(a[:, None, :] * et, axis=2)
    me = jnp.dot(m, wn_ref[d:], preferred_element_type=jnp.float32)
    o_ref[0] = hs + deg * hn + me + b_ref[...]


def kernel(H, A, E, N, W_self, W_nei, bias):
    B, Nn, D = H.shape
    De = E.shape[-1]
    Et = jnp.swapaxes(E, 2, 3)
    bias2 = bias[None, :]

    IB = Nn // 2
    grid = (B, Nn // IB)
    out = pl.pallas_call(
        functools.partial(_mp_body, d=D),
        grid=grid,
        in_specs=[
            pl.BlockSpec((1, IB, D), lambda a, q: (a, q, 0)),
            pl.BlockSpec((1, IB, Nn), lambda a, q: (a, q, 0)),
            pl.BlockSpec((1, IB, De, Nn), lambda a, q: (a, q, 0, 0)),
            pl.BlockSpec((D, D), lambda a, q: (0, 0)),
            pl.BlockSpec((D + De, D), lambda a, q: (0, 0)),
            pl.BlockSpec((1, D), lambda a, q: (0, 0)),
        ],
        out_specs=pl.BlockSpec((1, IB, D), lambda a, q: (a, q, 0)),
        out_shape=jax.ShapeDtypeStruct((B, Nn, D), jnp.float32),
        compiler_params=pltpu.CompilerParams(
            dimension_semantics=("arbitrary", "arbitrary"),
        ),
    )(H, A, Et, W_self, W_nei, bias2)
    return out


# speedup vs baseline: 1.4837x; 1.4837x over previous
import functools

import jax
import jax.numpy as jnp
from jax.experimental import pallas as pl
from jax.experimental.pallas import tpu as pltpu


def _mp_body(h_ref, a_ref, et_ref, ws_ref, wn_ref, b_ref, o_ref, *, d):
    h = h_ref[0]
    a = a_ref[0]
    et = et_ref[0]

    hs = jnp.dot(h, ws_ref[...], preferred_element_type=jnp.float32)
    hn = jnp.dot(h, wn_ref[:d], preferred_element_type=jnp.float32)
    deg = jnp.sum(a, axis=1, keepdims=True)
    m = jnp.sum(a[:, None, :] * et, axis=2)
    me = jnp.dot(m, wn_ref[d:], preferred_element_type=jnp.float32)
    o_ref[0] = hs + deg * hn + me + b_ref[...]


def kernel(H, A, E, N, W_self, W_nei, bias):
    B, Nn, D = H.shape
    De = E.shape[-1]
    Et = jnp.swapaxes(E, 2, 3)
    bias2 = bias[None, :]

    grid = (B,)
    out = pl.pallas_call(
        functools.partial(_mp_body, d=D),
        grid=grid,
        in_specs=[
            pl.BlockSpec((1, Nn, D), lambda a: (a, 0, 0)),
            pl.BlockSpec((1, Nn, Nn), lambda a: (a, 0, 0)),
            pl.BlockSpec((1, Nn, De, Nn), lambda a: (a, 0, 0, 0)),
            pl.BlockSpec((D, D), lambda a: (0, 0)),
            pl.BlockSpec((D + De, D), lambda a: (0, 0)),
            pl.BlockSpec((1, D), lambda a: (0, 0)),
        ],
        out_specs=pl.BlockSpec((1, Nn, D), lambda a: (a, 0, 0)),
        out_shape=jax.ShapeDtypeStruct((B, Nn, D), jnp.float32),
        compiler_params=pltpu.CompilerParams(
            dimension_semantics=("arbitrary",),
        ),
    )(H, A, Et, W_self, W_nei, bias2)
    return out


# speedup vs baseline: 1.9850x; 1.3379x over previous
import functools

import jax
import jax.numpy as jnp
from jax.experimental import pallas as pl
from jax.experimental.pallas import tpu as pltpu


def _mp_body(h_ref, a_ref, et_ref, ws_ref, wn_ref, b_ref, o_ref, *, d, bb, n):
    h = h_ref[...].reshape(bb * n, d)
    a = a_ref[...]
    et = et_ref[...]

    hs = jnp.dot(h, ws_ref[...], preferred_element_type=jnp.float32)
    hn = jnp.dot(h, wn_ref[:d], preferred_element_type=jnp.float32)
    deg = jnp.sum(a, axis=2).reshape(bb * n, 1)
    m = jnp.sum(a[:, :, None, :] * et, axis=3)
    me = jnp.dot(m.reshape(bb * n, et.shape[2]), wn_ref[d:],
                 preferred_element_type=jnp.float32)
    o_ref[...] = (hs + deg * hn + me + b_ref[...]).reshape(bb, n, d)


def kernel(H, A, E, N, W_self, W_nei, bias):
    B, Nn, D = H.shape
    De = E.shape[-1]
    Et = jnp.swapaxes(E, 2, 3)
    bias2 = bias[None, :]

    BB = 2
    grid = (B // BB,)
    out = pl.pallas_call(
        functools.partial(_mp_body, d=D, bb=BB, n=Nn),
        grid=grid,
        in_specs=[
            pl.BlockSpec((BB, Nn, D), lambda a: (a, 0, 0)),
            pl.BlockSpec((BB, Nn, Nn), lambda a: (a, 0, 0)),
            pl.BlockSpec((BB, Nn, De, Nn), lambda a: (a, 0, 0, 0)),
            pl.BlockSpec((D, D), lambda a: (0, 0)),
            pl.BlockSpec((D + De, D), lambda a: (0, 0)),
            pl.BlockSpec((1, D), lambda a: (0, 0)),
        ],
        out_specs=pl.BlockSpec((BB, Nn, D), lambda a: (a, 0, 0)),
        out_shape=jax.ShapeDtypeStruct((B, Nn, D), jnp.float32),
        compiler_params=pltpu.CompilerParams(
            dimension_semantics=("arbitrary",),
        ),
    )(H, A, Et, W_self, W_nei, bias2)
    return out


# speedup vs baseline: 2.3859x; 1.2020x over previous
import functools

import jax
import jax.numpy as jnp
from jax.experimental import pallas as pl
from jax.experimental.pallas import tpu as pltpu


def _mp_body(h_ref, a_ref, et_ref, ws_ref, wn_ref, b_ref, o_ref, *, d, bb, n):
    h = h_ref[...].reshape(bb * n, d)
    a = a_ref[...]
    et = et_ref[...]

    hs = jnp.dot(h, ws_ref[...], preferred_element_type=jnp.float32)
    hn = jnp.dot(h, wn_ref[:d], preferred_element_type=jnp.float32)
    deg = jnp.sum(a, axis=2).reshape(bb * n, 1)
    m = jnp.sum(a[:, :, None, :] * et, axis=3)
    me = jnp.dot(m.reshape(bb * n, et.shape[2]), wn_ref[d:],
                 preferred_element_type=jnp.float32)
    o_ref[...] = (hs + deg * hn + me + b_ref[...]).reshape(bb, n, d)


def kernel(H, A, E, N, W_self, W_nei, bias):
    B, Nn, D = H.shape
    De = E.shape[-1]
    Et = jnp.swapaxes(E, 2, 3)
    bias2 = bias[None, :]

    BB = 4
    grid = (B // BB,)
    out = pl.pallas_call(
        functools.partial(_mp_body, d=D, bb=BB, n=Nn),
        grid=grid,
        in_specs=[
            pl.BlockSpec((BB, Nn, D), lambda a: (a, 0, 0)),
            pl.BlockSpec((BB, Nn, Nn), lambda a: (a, 0, 0)),
            pl.BlockSpec((BB, Nn, De, Nn), lambda a: (a, 0, 0, 0)),
            pl.BlockSpec((D, D), lambda a: (0, 0)),
            pl.BlockSpec((D + De, D), lambda a: (0, 0)),
            pl.BlockSpec((1, D), lambda a: (0, 0)),
        ],
        out_specs=pl.BlockSpec((BB, Nn, D), lambda a: (a, 0, 0)),
        out_shape=jax.ShapeDtypeStruct((B, Nn, D), jnp.float32),
        compiler_params=pltpu.CompilerParams(
            dimension_semantics=("arbitrary",),
        ),
    )(H, A, Et, W_self, W_nei, bias2)
    return out


# speedup vs baseline: 2.5536x; 1.0703x over previous
import functools

import jax
import jax.numpy as jnp
from jax.experimental import pallas as pl
from jax.experimental.pallas import tpu as pltpu


def _mp_body(h_ref, a_ref, et_ref, ws_ref, wn_ref, b_ref, o_ref, *, d, bb, n):
    h = h_ref[...].reshape(bb * n, d)
    a = a_ref[...]
    et = et_ref[...]

    hs = jnp.dot(h, ws_ref[...], preferred_element_type=jnp.float32)
    hn = jnp.dot(h, wn_ref[:d], preferred_element_type=jnp.float32)
    deg = jnp.sum(a, axis=2).reshape(bb * n, 1)
    m = jnp.sum(a[:, :, None, :] * et, axis=3)
    me = jnp.dot(m.reshape(bb * n, et.shape[2]), wn_ref[d:],
                 preferred_element_type=jnp.float32)
    o_ref[...] = (hs + deg * hn + me + b_ref[...]).reshape(bb, n, d)


def kernel(H, A, E, N, W_self, W_nei, bias):
    B, Nn, D = H.shape
    De = E.shape[-1]
    Et = jnp.swapaxes(E, 2, 3)
    bias2 = bias[None, :]

    BB = 8
    grid = (B // BB,)
    out = pl.pallas_call(
        functools.partial(_mp_body, d=D, bb=BB, n=Nn),
        grid=grid,
        in_specs=[
            pl.BlockSpec((BB, Nn, D), lambda a: (a, 0, 0)),
            pl.BlockSpec((BB, Nn, Nn), lambda a: (a, 0, 0)),
            pl.BlockSpec((BB, Nn, De, Nn), lambda a: (a, 0, 0, 0)),
            pl.BlockSpec((D, D), lambda a: (0, 0)),
            pl.BlockSpec((D + De, D), lambda a: (0, 0)),
            pl.BlockSpec((1, D), lambda a: (0, 0)),
        ],
        out_specs=pl.BlockSpec((BB, Nn, D), lambda a: (a, 0, 0)),
        out_shape=jax.ShapeDtypeStruct((B, Nn, D), jnp.float32),
        compiler_params=pltpu.CompilerParams(
            dimension_semantics=("arbitrary",),
        ),
    )(H, A, Et, W_self, W_nei, bias2)
    return out
